# baseline (speedup 1.0000x reference)
import jax
import jax.numpy as jnp
from jax.experimental import pallas as pl


def _copy_block(x_ref, o_ref):
    o_ref[...] = x_ref[...]


def kernel(X, arm, edge_index):
    n, d = X.shape
    rows = 1000
    return pl.pallas_call(
        _copy_block,
        grid=(n // rows,),
        in_specs=[pl.BlockSpec((rows, d), lambda i: (i, 0))],
        out_specs=pl.BlockSpec((rows, d), lambda i: (i, 0)),
        out_shape=jax.ShapeDtypeStruct((n, d), X.dtype),
    )(X)
